# baseline (speedup 1.0000x reference)
import functools

import jax
import jax.numpy as jnp
from jax.experimental import pallas as pl
from jax.experimental.pallas import tpu as pltpu
from jax.experimental.pallas import tpu_sc as plsc

_GATHER_WINDOW = 128
_MM_BLOCK = 512
_LANE = 128
_W = 256


def _tc_build_table(embt, v_pad):
    d, v = embt.shape

    def build_kernel(x_ref, o_ref):
        xt = jnp.swapaxes(x_ref[...], 0, 1)
        zeros = jnp.zeros((_LANE, 2 * _W - d), jnp.float32)
        xt = jnp.concatenate([xt, zeros], axis=1)
        lo = xt[:, :_W]
        hi = xt[:, _W:]
        packed = pltpu.pack_elementwise([lo, hi], packed_dtype=jnp.bfloat16)
        o_ref[...] = pltpu.bitcast(packed, jnp.float32)

    n_blk = v_pad // _LANE
    return pl.pallas_call(
        build_kernel,
        grid=(n_blk,),
        in_specs=[pl.BlockSpec((d, _LANE), lambda i: (0, i))],
        out_specs=pl.BlockSpec((_LANE, _W), lambda i: (i, 0)),
        out_shape=jax.ShapeDtypeStruct((v_pad, _W), jnp.float32),
    )(embt)


def _sc_gather(table, idx):
    n = idx.shape[1]

    mesh = plsc.VectorSubcoreMesh(core_axis_name="core", subcore_axis_name="subcore")

    @functools.partial(
        pl.kernel,
        out_type=jax.ShapeDtypeStruct((n, _W), jnp.float32),
        mesh=mesh,
    )
    def gather_kernel(t_hbm, i_hbm, o_hbm):
        def body(i_vmem, o_vmem):
            pltpu.sync_copy(t_hbm.at[i_vmem.at[0]], o_vmem)

        pltpu.emit_pipeline(
            body,
            grid=(n // _GATHER_WINDOW,),
            in_specs=[pl.BlockSpec((1, _GATHER_WINDOW), lambda i: (0, i))],
            out_specs=[pl.BlockSpec((_GATHER_WINDOW, _W), lambda i: (i, 0))],
            core_axis_name=("core", "subcore"),
            dimension_semantics=(pltpu.PARALLEL,),
        )(i_hbm, o_hbm)

    return gather_kernel(table, idx)


def _tc_project(e, wp, bq2):
    n = e.shape[0]
    h = wp.shape[1]

    def mm_kernel(e_ref, w0_ref, w1_ref, w2_ref, b_ref, o_ref):
        w = pltpu.bitcast(e_ref[...], jnp.int32)
        elo = pltpu.unpack_elementwise(
            w, index=0, packed_dtype=jnp.bfloat16, unpacked_dtype=jnp.float32
        )
        ehi = pltpu.unpack_elementwise(
            w, index=1, packed_dtype=jnp.bfloat16, unpacked_dtype=jnp.float32
        )
        e0 = elo[:, :_LANE].astype(jnp.bfloat16)
        e1 = elo[:, _LANE:].astype(jnp.bfloat16)
        e2 = ehi[:, :_LANE].astype(jnp.bfloat16)
        acc = jnp.dot(e0, w0_ref[...], preferred_element_type=jnp.float32)
        acc += jnp.dot(e1, w1_ref[...], preferred_element_type=jnp.float32)
        acc += jnp.dot(e2, w2_ref[...], preferred_element_type=jnp.float32)
        o_ref[...] = acc + b_ref[...]

    return pl.pallas_call(
        mm_kernel,
        grid=(n // _MM_BLOCK,),
        in_specs=[
            pl.BlockSpec((_MM_BLOCK, _W), lambda i: (i, 0)),
            pl.BlockSpec((_LANE, h), lambda i: (0, 0)),
            pl.BlockSpec((_LANE, h), lambda i: (1, 0)),
            pl.BlockSpec((_LANE, h), lambda i: (2, 0)),
            pl.BlockSpec((1, h), lambda i: (0, 0)),
        ],
        out_specs=pl.BlockSpec((_MM_BLOCK, h), lambda i: (i, 0)),
        out_shape=jax.ShapeDtypeStruct((n, h), jnp.float32),
    )(e, wp, wp, wp, bq2)


def kernel(ques, emb, Wq, bq):
    b, l = ques.shape
    v, d = emb.shape
    h = Wq.shape[1]
    v_pad = ((v + _LANE - 1) // _LANE) * _LANE
    idx = ques.T.reshape(1, b * l).astype(jnp.int32)
    embt = emb.T
    table = _tc_build_table(embt, v_pad)
    wp = jnp.concatenate(
        [Wq.astype(jnp.bfloat16), jnp.zeros((3 * _LANE - d, h), jnp.bfloat16)], axis=0
    )
    e = _sc_gather(table, idx)
    out = _tc_project(e, wp, bq.reshape(1, h))
    return out.reshape(l, b, h).transpose(1, 0, 2)


# speedup vs baseline: 3.5185x; 3.5185x over previous
import functools

import jax
import jax.numpy as jnp
from jax.experimental import pallas as pl
from jax.experimental.pallas import tpu as pltpu
from jax.experimental.pallas import tpu_sc as plsc

_GATHER_WINDOW = 128
_MM_BLOCK = 512
_LANE = 128
_W = 256
_TB = 2048


def _tc_build_table(embt, v_pad):
    d, v = embt.shape

    def build_kernel(x_ref, o_ref):
        xt = jnp.swapaxes(x_ref[...], 0, 1)
        zeros = jnp.zeros((_TB, 2 * _W - d), jnp.float32)
        xt = jnp.concatenate([xt, zeros], axis=1)
        lo = xt[:, :_W]
        hi = xt[:, _W:]
        packed = pltpu.pack_elementwise([lo, hi], packed_dtype=jnp.bfloat16)
        o_ref[...] = pltpu.bitcast(packed, jnp.float32)

    n_blk = v_pad // _TB
    return pl.pallas_call(
        build_kernel,
        grid=(n_blk,),
        in_specs=[pl.BlockSpec((d, _TB), lambda i: (0, i))],
        out_specs=pl.BlockSpec((_TB, _W), lambda i: (i, 0)),
        out_shape=jax.ShapeDtypeStruct((v_pad, _W), jnp.float32),
    )(embt)


def _sc_gather(table, idx):
    n = idx.shape[1]

    mesh = plsc.VectorSubcoreMesh(core_axis_name="core", subcore_axis_name="subcore")

    @functools.partial(
        pl.kernel,
        out_type=jax.ShapeDtypeStruct((n, _W), jnp.float32),
        mesh=mesh,
    )
    def gather_kernel(t_hbm, i_hbm, o_hbm):
        def body(i_vmem, o_vmem):
            pltpu.sync_copy(t_hbm.at[i_vmem.at[0]], o_vmem)

        pltpu.emit_pipeline(
            body,
            grid=(n // _GATHER_WINDOW,),
            in_specs=[pl.BlockSpec((1, _GATHER_WINDOW), lambda i: (0, i))],
            out_specs=[pl.BlockSpec((_GATHER_WINDOW, _W), lambda i: (i, 0))],
            core_axis_name=("core", "subcore"),
            dimension_semantics=(pltpu.PARALLEL,),
        )(i_hbm, o_hbm)

    return gather_kernel(table, idx)


def _tc_project(e, wp, bq2):
    n = e.shape[0]
    h = wp.shape[1]

    def mm_kernel(e_ref, w0_ref, w1_ref, w2_ref, b_ref, o_ref):
        w = pltpu.bitcast(e_ref[...], jnp.int32)
        elo = pltpu.unpack_elementwise(
            w, index=0, packed_dtype=jnp.bfloat16, unpacked_dtype=jnp.float32
        )
        ehi = pltpu.unpack_elementwise(
            w, index=1, packed_dtype=jnp.bfloat16, unpacked_dtype=jnp.float32
        )
        e0 = elo[:, :_LANE].astype(jnp.bfloat16)
        e1 = elo[:, _LANE:].astype(jnp.bfloat16)
        e2 = ehi[:, :_LANE].astype(jnp.bfloat16)
        acc = jnp.dot(e0, w0_ref[...], preferred_element_type=jnp.float32)
        acc += jnp.dot(e1, w1_ref[...], preferred_element_type=jnp.float32)
        acc += jnp.dot(e2, w2_ref[...], preferred_element_type=jnp.float32)
        o_ref[...] = acc + b_ref[...]

    return pl.pallas_call(
        mm_kernel,
        grid=(n // _MM_BLOCK,),
        in_specs=[
            pl.BlockSpec((_MM_BLOCK, _W), lambda i: (i, 0)),
            pl.BlockSpec((_LANE, h), lambda i: (0, 0)),
            pl.BlockSpec((_LANE, h), lambda i: (1, 0)),
            pl.BlockSpec((_LANE, h), lambda i: (2, 0)),
            pl.BlockSpec((1, h), lambda i: (0, 0)),
        ],
        out_specs=pl.BlockSpec((_MM_BLOCK, h), lambda i: (i, 0)),
        out_shape=jax.ShapeDtypeStruct((n, h), jnp.float32),
    )(e, wp, wp, wp, bq2)


def kernel(ques, emb, Wq, bq):
    b, l = ques.shape
    v, d = emb.shape
    h = Wq.shape[1]
    v_pad = ((v + _TB - 1) // _TB) * _TB
    idx = ques.T.reshape(1, b * l).astype(jnp.int32)
    embt = emb.T
    table = _tc_build_table(embt, v_pad)
    wp = jnp.concatenate(
        [Wq.astype(jnp.bfloat16), jnp.zeros((3 * _LANE - d, h), jnp.bfloat16)], axis=0
    )
    e = _sc_gather(table, idx)
    out = _tc_project(e, wp, bq.reshape(1, h))
    return out.reshape(l, b, h).transpose(1, 0, 2)


# speedup vs baseline: 4.0518x; 1.1516x over previous
import functools

import jax
import jax.numpy as jnp
from jax.experimental import pallas as pl
from jax.experimental.pallas import tpu as pltpu
from jax.experimental.pallas import tpu_sc as plsc

_GATHER_WINDOW = 128
_MM_BLOCK = 1024
_LANE = 128
_W = 256
_TB = 4096


def _tc_build_table(embt, v_pad):
    d, v = embt.shape

    def build_kernel(x_ref, o_ref):
        xt = jnp.swapaxes(x_ref[...], 0, 1)
        zeros = jnp.zeros((_TB, 2 * _W - d), jnp.float32)
        xt = jnp.concatenate([xt, zeros], axis=1)
        lo = xt[:, :_W]
        hi = xt[:, _W:]
        packed = pltpu.pack_elementwise([lo, hi], packed_dtype=jnp.bfloat16)
        o_ref[...] = pltpu.bitcast(packed, jnp.float32)

    n_blk = v_pad // _TB
    return pl.pallas_call(
        build_kernel,
        grid=(n_blk,),
        in_specs=[pl.BlockSpec((d, _TB), lambda i: (0, i))],
        out_specs=pl.BlockSpec((_TB, _W), lambda i: (i, 0)),
        out_shape=jax.ShapeDtypeStruct((v_pad, _W), jnp.float32),
    )(embt)


def _sc_gather(table, idx):
    n = idx.shape[1]

    mesh = plsc.VectorSubcoreMesh(core_axis_name="core", subcore_axis_name="subcore")

    @functools.partial(
        pl.kernel,
        out_type=jax.ShapeDtypeStruct((n, _W), jnp.float32),
        mesh=mesh,
    )
    def gather_kernel(t_hbm, i_hbm, o_hbm):
        def body(i_vmem, o_vmem):
            pltpu.sync_copy(t_hbm.at[i_vmem.at[0]], o_vmem)

        pltpu.emit_pipeline(
            body,
            grid=(n // _GATHER_WINDOW,),
            in_specs=[pl.BlockSpec((1, _GATHER_WINDOW), lambda i: (0, i))],
            out_specs=[pl.BlockSpec((_GATHER_WINDOW, _W), lambda i: (i, 0))],
            core_axis_name=("core", "subcore"),
            dimension_semantics=(pltpu.PARALLEL,),
        )(i_hbm, o_hbm)

    return gather_kernel(table, idx)


def _tc_project(e, wp, bq2):
    n = e.shape[0]
    h = wp.shape[1]

    def mm_kernel(e_ref, w0_ref, w1_ref, w2_ref, b_ref, o_ref):
        w = pltpu.bitcast(e_ref[...], jnp.int32)
        elo = pltpu.unpack_elementwise(
            w, index=0, packed_dtype=jnp.bfloat16, unpacked_dtype=jnp.float32
        )
        ehi = pltpu.unpack_elementwise(
            w, index=1, packed_dtype=jnp.bfloat16, unpacked_dtype=jnp.float32
        )
        e0 = elo[:, :_LANE].astype(jnp.bfloat16)
        e1 = elo[:, _LANE:].astype(jnp.bfloat16)
        e2 = ehi[:, :_LANE].astype(jnp.bfloat16)
        acc = jnp.dot(e0, w0_ref[...], preferred_element_type=jnp.float32)
        acc += jnp.dot(e1, w1_ref[...], preferred_element_type=jnp.float32)
        acc += jnp.dot(e2, w2_ref[...], preferred_element_type=jnp.float32)
        o_ref[...] = acc + b_ref[...]

    return pl.pallas_call(
        mm_kernel,
        grid=(n // _MM_BLOCK,),
        in_specs=[
            pl.BlockSpec((_MM_BLOCK, _W), lambda i: (i, 0)),
            pl.BlockSpec((_LANE, h), lambda i: (0, 0)),
            pl.BlockSpec((_LANE, h), lambda i: (1, 0)),
            pl.BlockSpec((_LANE, h), lambda i: (2, 0)),
            pl.BlockSpec((1, h), lambda i: (0, 0)),
        ],
        out_specs=pl.BlockSpec((_MM_BLOCK, h), lambda i: (i, 0)),
        out_shape=jax.ShapeDtypeStruct((n, h), jnp.float32),
    )(e, wp, wp, wp, bq2)


def kernel(ques, emb, Wq, bq):
    b, l = ques.shape
    v, d = emb.shape
    h = Wq.shape[1]
    v_pad = ((v + _TB - 1) // _TB) * _TB
    idx = ques.T.reshape(1, b * l).astype(jnp.int32)
    embt = emb.T
    table = _tc_build_table(embt, v_pad)
    wp = jnp.concatenate(
        [Wq.astype(jnp.bfloat16), jnp.zeros((3 * _LANE - d, h), jnp.bfloat16)], axis=0
    )
    e = _sc_gather(table, idx)
    out = _tc_project(e, wp, bq.reshape(1, h))
    return out.reshape(l, b, h).transpose(1, 0, 2)


# speedup vs baseline: 4.3203x; 1.0663x over previous
import functools

import jax
import jax.numpy as jnp
from jax.experimental import pallas as pl
from jax.experimental.pallas import tpu as pltpu
from jax.experimental.pallas import tpu_sc as plsc

_GATHER_WINDOW = 128
_MM_BLOCK = 2048
_LANE = 128
_W = 256
_TB = 8192


def _tc_build_table(embt, v_pad):
    d, v = embt.shape

    def build_kernel(x_ref, o_ref):
        xt = jnp.swapaxes(x_ref[...], 0, 1)
        zeros = jnp.zeros((_TB, 2 * _W - d), jnp.float32)
        xt = jnp.concatenate([xt, zeros], axis=1)
        lo = xt[:, :_W]
        hi = xt[:, _W:]
        packed = pltpu.pack_elementwise([lo, hi], packed_dtype=jnp.bfloat16)
        o_ref[...] = pltpu.bitcast(packed, jnp.float32)

    n_blk = v_pad // _TB
    return pl.pallas_call(
        build_kernel,
        grid=(n_blk,),
        in_specs=[pl.BlockSpec((d, _TB), lambda i: (0, i))],
        out_specs=pl.BlockSpec((_TB, _W), lambda i: (i, 0)),
        out_shape=jax.ShapeDtypeStruct((v_pad, _W), jnp.float32),
    )(embt)


def _sc_gather(table, idx):
    n = idx.shape[1]

    mesh = plsc.VectorSubcoreMesh(core_axis_name="core", subcore_axis_name="subcore")

    @functools.partial(
        pl.kernel,
        out_type=jax.ShapeDtypeStruct((n, _W), jnp.float32),
        mesh=mesh,
    )
    def gather_kernel(t_hbm, i_hbm, o_hbm):
        def body(i_vmem, o_vmem):
            pltpu.sync_copy(t_hbm.at[i_vmem.at[0]], o_vmem)

        pltpu.emit_pipeline(
            body,
            grid=(n // _GATHER_WINDOW,),
            in_specs=[pl.BlockSpec((1, _GATHER_WINDOW), lambda i: (0, i))],
            out_specs=[pl.BlockSpec((_GATHER_WINDOW, _W), lambda i: (i, 0))],
            core_axis_name=("core", "subcore"),
            dimension_semantics=(pltpu.PARALLEL,),
        )(i_hbm, o_hbm)

    return gather_kernel(table, idx)


def _tc_project(e, wp, bq2):
    n = e.shape[0]
    h = wp.shape[1]

    def mm_kernel(e_ref, w0_ref, w1_ref, w2_ref, b_ref, o_ref):
        w = pltpu.bitcast(e_ref[...], jnp.int32)
        elo = pltpu.unpack_elementwise(
            w, index=0, packed_dtype=jnp.bfloat16, unpacked_dtype=jnp.float32
        )
        ehi = pltpu.unpack_elementwise(
            w, index=1, packed_dtype=jnp.bfloat16, unpacked_dtype=jnp.float32
        )
        e0 = elo[:, :_LANE].astype(jnp.bfloat16)
        e1 = elo[:, _LANE:].astype(jnp.bfloat16)
        e2 = ehi[:, :_LANE].astype(jnp.bfloat16)
        acc = jnp.dot(e0, w0_ref[...], preferred_element_type=jnp.float32)
        acc += jnp.dot(e1, w1_ref[...], preferred_element_type=jnp.float32)
        acc += jnp.dot(e2, w2_ref[...], preferred_element_type=jnp.float32)
        o_ref[...] = acc + b_ref[...]

    return pl.pallas_call(
        mm_kernel,
        grid=(n // _MM_BLOCK,),
        in_specs=[
            pl.BlockSpec((_MM_BLOCK, _W), lambda i: (i, 0)),
            pl.BlockSpec((_LANE, h), lambda i: (0, 0)),
            pl.BlockSpec((_LANE, h), lambda i: (1, 0)),
            pl.BlockSpec((_LANE, h), lambda i: (2, 0)),
            pl.BlockSpec((1, h), lambda i: (0, 0)),
        ],
        out_specs=pl.BlockSpec((_MM_BLOCK, h), lambda i: (i, 0)),
        out_shape=jax.ShapeDtypeStruct((n, h), jnp.float32),
    )(e, wp, wp, wp, bq2)


def kernel(ques, emb, Wq, bq):
    b, l = ques.shape
    v, d = emb.shape
    h = Wq.shape[1]
    v_pad = ((v + _TB - 1) // _TB) * _TB
    idx = ques.T.reshape(1, b * l).astype(jnp.int32)
    embt = emb.T
    table = _tc_build_table(embt, v_pad)
    wp = jnp.concatenate(
        [Wq.astype(jnp.bfloat16), jnp.zeros((3 * _LANE - d, h), jnp.bfloat16)], axis=0
    )
    e = _sc_gather(table, idx)
    out = _tc_project(e, wp, bq.reshape(1, h))
    return out.reshape(l, b, h).transpose(1, 0, 2)


# speedup vs baseline: 4.3827x; 1.0145x over previous
import functools

import jax
import jax.numpy as jnp
from jax.experimental import pallas as pl
from jax.experimental.pallas import tpu as pltpu
from jax.experimental.pallas import tpu_sc as plsc

_GATHER_WINDOW = 128
_MM_BLOCK = 4096
_LANE = 128
_W = 256
_TB = 8192


def _tc_build_table(embt, v_pad):
    d, v = embt.shape

    def build_kernel(x_ref, o_ref):
        xt = jnp.swapaxes(x_ref[...], 0, 1)
        zeros = jnp.zeros((_TB, 2 * _W - d), jnp.float32)
        xt = jnp.concatenate([xt, zeros], axis=1)
        lo = xt[:, :_W]
        hi = xt[:, _W:]
        packed = pltpu.pack_elementwise([lo, hi], packed_dtype=jnp.bfloat16)
        o_ref[...] = pltpu.bitcast(packed, jnp.float32)

    n_blk = v_pad // _TB
    return pl.pallas_call(
        build_kernel,
        grid=(n_blk,),
        in_specs=[pl.BlockSpec((d, _TB), lambda i: (0, i))],
        out_specs=pl.BlockSpec((_TB, _W), lambda i: (i, 0)),
        out_shape=jax.ShapeDtypeStruct((v_pad, _W), jnp.float32),
    )(embt)


def _sc_gather(table, idx):
    n = idx.shape[1]

    mesh = plsc.VectorSubcoreMesh(core_axis_name="core", subcore_axis_name="subcore")

    @functools.partial(
        pl.kernel,
        out_type=jax.ShapeDtypeStruct((n, _W), jnp.float32),
        mesh=mesh,
    )
    def gather_kernel(t_hbm, i_hbm, o_hbm):
        def body(i_vmem, o_vmem):
            pltpu.sync_copy(t_hbm.at[i_vmem.at[0]], o_vmem)

        pltpu.emit_pipeline(
            body,
            grid=(n // _GATHER_WINDOW,),
            in_specs=[pl.BlockSpec((1, _GATHER_WINDOW), lambda i: (0, i))],
            out_specs=[pl.BlockSpec((_GATHER_WINDOW, _W), lambda i: (i, 0))],
            core_axis_name=("core", "subcore"),
            dimension_semantics=(pltpu.PARALLEL,),
        )(i_hbm, o_hbm)

    return gather_kernel(table, idx)


def _tc_project(e, wp, bq2):
    n = e.shape[0]
    h = wp.shape[1]

    def mm_kernel(e_ref, w0_ref, w1_ref, w2_ref, b_ref, o_ref):
        w = pltpu.bitcast(e_ref[...], jnp.int32)
        elo = pltpu.unpack_elementwise(
            w, index=0, packed_dtype=jnp.bfloat16, unpacked_dtype=jnp.float32
        )
        ehi = pltpu.unpack_elementwise(
            w, index=1, packed_dtype=jnp.bfloat16, unpacked_dtype=jnp.float32
        )
        e0 = elo[:, :_LANE].astype(jnp.bfloat16)
        e1 = elo[:, _LANE:].astype(jnp.bfloat16)
        e2 = ehi[:, :_LANE].astype(jnp.bfloat16)
        acc = jnp.dot(e0, w0_ref[...], preferred_element_type=jnp.float32)
        acc += jnp.dot(e1, w1_ref[...], preferred_element_type=jnp.float32)
        acc += jnp.dot(e2, w2_ref[...], preferred_element_type=jnp.float32)
        o_ref[...] = acc + b_ref[...]

    return pl.pallas_call(
        mm_kernel,
        grid=(n // _MM_BLOCK,),
        in_specs=[
            pl.BlockSpec((_MM_BLOCK, _W), lambda i: (i, 0)),
            pl.BlockSpec((_LANE, h), lambda i: (0, 0)),
            pl.BlockSpec((_LANE, h), lambda i: (1, 0)),
            pl.BlockSpec((_LANE, h), lambda i: (2, 0)),
            pl.BlockSpec((1, h), lambda i: (0, 0)),
        ],
        out_specs=pl.BlockSpec((_MM_BLOCK, h), lambda i: (i, 0)),
        out_shape=jax.ShapeDtypeStruct((n, h), jnp.float32),
    )(e, wp, wp, wp, bq2)


def kernel(ques, emb, Wq, bq):
    b, l = ques.shape
    v, d = emb.shape
    h = Wq.shape[1]
    v_pad = ((v + _TB - 1) // _TB) * _TB
    idx = ques.T.reshape(1, b * l).astype(jnp.int32)
    embt = emb.T
    table = _tc_build_table(embt, v_pad)
    wp = jnp.concatenate(
        [Wq.astype(jnp.bfloat16), jnp.zeros((3 * _LANE - d, h), jnp.bfloat16)], axis=0
    )
    e = _sc_gather(table, idx)
    out = _tc_project(e, wp, bq.reshape(1, h))
    return out.reshape(l, b, h).transpose(1, 0, 2)


# speedup vs baseline: 4.4011x; 1.0042x over previous
import functools

import jax
import jax.numpy as jnp
from jax.experimental import pallas as pl
from jax.experimental.pallas import tpu as pltpu
from jax.experimental.pallas import tpu_sc as plsc

_GATHER_WINDOW = 128
_MM_BLOCK = 4096
_LANE = 128
_W = 256
_TB = 12288


def _tc_build_table(embt, v_pad):
    d, v = embt.shape

    def build_kernel(x_ref, o_ref):
        xt = jnp.swapaxes(x_ref[...], 0, 1)
        zeros = jnp.zeros((_TB, 2 * _W - d), jnp.float32)
        xt = jnp.concatenate([xt, zeros], axis=1)
        lo = xt[:, :_W]
        hi = xt[:, _W:]
        packed = pltpu.pack_elementwise([lo, hi], packed_dtype=jnp.bfloat16)
        o_ref[...] = pltpu.bitcast(packed, jnp.float32)

    n_blk = v_pad // _TB
    return pl.pallas_call(
        build_kernel,
        grid=(n_blk,),
        in_specs=[pl.BlockSpec((d, _TB), lambda i: (0, i))],
        out_specs=pl.BlockSpec((_TB, _W), lambda i: (i, 0)),
        out_shape=jax.ShapeDtypeStruct((v_pad, _W), jnp.float32),
    )(embt)


def _sc_gather(table, idx):
    n = idx.shape[1]

    mesh = plsc.VectorSubcoreMesh(core_axis_name="core", subcore_axis_name="subcore")

    @functools.partial(
        pl.kernel,
        out_type=jax.ShapeDtypeStruct((n, _W), jnp.float32),
        mesh=mesh,
    )
    def gather_kernel(t_hbm, i_hbm, o_hbm):
        def body(i_vmem, o_vmem):
            pltpu.sync_copy(t_hbm.at[i_vmem.at[0]], o_vmem)

        pltpu.emit_pipeline(
            body,
            grid=(n // _GATHER_WINDOW,),
            in_specs=[pl.BlockSpec((1, _GATHER_WINDOW), lambda i: (0, i))],
            out_specs=[pl.BlockSpec((_GATHER_WINDOW, _W), lambda i: (i, 0))],
            core_axis_name=("core", "subcore"),
            dimension_semantics=(pltpu.PARALLEL,),
        )(i_hbm, o_hbm)

    return gather_kernel(table, idx)


def _tc_project(e, wp, bq2):
    n = e.shape[0]
    h = wp.shape[1]

    def mm_kernel(e_ref, w0_ref, w1_ref, w2_ref, b_ref, o_ref):
        w = pltpu.bitcast(e_ref[...], jnp.int32)
        elo = pltpu.unpack_elementwise(
            w, index=0, packed_dtype=jnp.bfloat16, unpacked_dtype=jnp.float32
        )
        ehi = pltpu.unpack_elementwise(
            w, index=1, packed_dtype=jnp.bfloat16, unpacked_dtype=jnp.float32
        )
        e0 = elo[:, :_LANE].astype(jnp.bfloat16)
        e1 = elo[:, _LANE:].astype(jnp.bfloat16)
        e2 = ehi[:, :_LANE].astype(jnp.bfloat16)
        acc = jnp.dot(e0, w0_ref[...], preferred_element_type=jnp.float32)
        acc += jnp.dot(e1, w1_ref[...], preferred_element_type=jnp.float32)
        acc += jnp.dot(e2, w2_ref[...], preferred_element_type=jnp.float32)
        o_ref[...] = acc + b_ref[...]

    return pl.pallas_call(
        mm_kernel,
        grid=(n // _MM_BLOCK,),
        in_specs=[
            pl.BlockSpec((_MM_BLOCK, _W), lambda i: (i, 0)),
            pl.BlockSpec((_LANE, h), lambda i: (0, 0)),
            pl.BlockSpec((_LANE, h), lambda i: (1, 0)),
            pl.BlockSpec((_LANE, h), lambda i: (2, 0)),
            pl.BlockSpec((1, h), lambda i: (0, 0)),
        ],
        out_specs=pl.BlockSpec((_MM_BLOCK, h), lambda i: (i, 0)),
        out_shape=jax.ShapeDtypeStruct((n, h), jnp.float32),
    )(e, wp, wp, wp, bq2)


def kernel(ques, emb, Wq, bq):
    b, l = ques.shape
    v, d = emb.shape
    h = Wq.shape[1]
    v_pad = ((v + _TB - 1) // _TB) * _TB
    idx = ques.T.reshape(1, b * l).astype(jnp.int32)
    embt = emb.T
    table = _tc_build_table(embt, v_pad)
    wp = jnp.concatenate(
        [Wq.astype(jnp.bfloat16), jnp.zeros((3 * _LANE - d, h), jnp.bfloat16)], axis=0
    )
    e = _sc_gather(table, idx)
    out = _tc_project(e, wp, bq.reshape(1, h))
    return out.reshape(l, b, h).transpose(1, 0, 2)
